# baseline (speedup 1.0000x reference)
import functools

import jax
import jax.numpy as jnp
from jax import lax
from jax.experimental import pallas as pl
from jax.experimental.pallas import tpu as pltpu
from jax.experimental.pallas import tpu_sc as plsc


def _add_block(x_ref, pos_ref, o_ref):
    o_ref[...] = x_ref[...] + pos_ref[...]


def _kernel_tc(X, pos_table):
    B, S, D = X.shape
    bs = 2048
    grid = (S // bs, B)
    out = pl.pallas_call(
        _add_block,
        grid=grid,
        in_specs=[
            pl.BlockSpec((1, bs, D), lambda s, b: (b, s, 0)),
            pl.BlockSpec((bs, D), lambda s, b: (s, 0)),
        ],
        out_specs=pl.BlockSpec((1, bs, D), lambda s, b: (b, s, 0)),
        out_shape=jax.ShapeDtypeStruct((B, S, D), X.dtype),
    )(X, pos_table[:S])
    return out


_NC, _NS, _NL = 2, 16, 16
_NW = _NC * _NS
_CRW = 8


def _sc_body(x_hbm, pos_hbm, o_hbm, *scratch):
    xbs = scratch[0:8]
    pbs = scratch[8:10]
    sxl = scratch[10:18]
    sxs = scratch[18:26]
    spl = scratch[26:28]

    w = lax.axis_index("c") * _NS + lax.axis_index("s")
    B, S, D = x_hbm.shape
    rows = S // _NW
    r0 = w * rows
    nsc = rows // _CRW

    def xload(sc, b, xb, sem):
        r = r0 + sc * _CRW
        pltpu.make_async_copy(x_hbm.at[b, pl.ds(r, _CRW), :], xb, sem).start()

    def xload_wait(xb, sem):
        pltpu.make_async_copy(x_hbm.at[0, pl.ds(r0, _CRW), :], xb, sem).wait()

    def pload(sc, pb, sem):
        r = r0 + sc * _CRW
        pltpu.make_async_copy(pos_hbm.at[pl.ds(r, _CRW), :], pb, sem).start()

    def pload_wait(pb, sem):
        pltpu.make_async_copy(pos_hbm.at[pl.ds(r0, _CRW), :], pb, sem).wait()

    def xstore(sc, b, xb, sem):
        r = r0 + sc * _CRW
        pltpu.make_async_copy(xb, o_hbm.at[b, pl.ds(r, _CRW), :], sem).start()

    def xstore_wait(xb, sem):
        pltpu.make_async_copy(xb, o_hbm.at[0, pl.ds(r0, _CRW), :], sem).wait()

    def compute(xb, pb):
        @plsc.parallel_loop(0, _CRW * D, step=_NL, unroll=8)
        def add(v):
            r = v // D
            sl = pl.ds(v % D, _NL)
            xb[r, sl] = xb[r, sl] + pb[r, sl]

    def half_pass(sc, pre, base, pb, psem):
        pload_wait(pb, psem)
        for b in range(4):
            xload_wait(xbs[base + b], sxl[base + b])
            compute(xbs[base + b], pb)
            xstore(sc, b, xbs[base + b], sxs[base + b])
        for b in range(4):
            xstore_wait(xbs[base + b], sxs[base + b])
            xload(pre, b, xbs[base + b], sxl[base + b])
        pload(pre, pb, psem)

    pload(0, pbs[0], spl[0])
    pload(1, pbs[1], spl[1])
    for b in range(4):
        xload(0, b, xbs[b], sxl[b])
        xload(1, b, xbs[4 + b], sxl[4 + b])

    def body(i, carry):
        sc0 = i * 2
        pre0 = lax.min(sc0 + 2, nsc - 2)
        pre1 = lax.min(sc0 + 3, nsc - 1)
        half_pass(sc0, pre0, 0, pbs[0], spl[0])
        half_pass(sc0 + 1, pre1, 4, pbs[1], spl[1])
        return carry

    lax.fori_loop(0, nsc // 2, body, 0)

    for j in range(8):
        xload_wait(xbs[j], sxl[j])
    pload_wait(pbs[0], spl[0])
    pload_wait(pbs[1], spl[1])


def _kernel_sc(X, pos_table):
    B, S, D = X.shape
    mesh = plsc.VectorSubcoreMesh(core_axis_name="c", subcore_axis_name="s")
    k = functools.partial(
        pl.kernel,
        mesh=mesh,
        out_type=jax.ShapeDtypeStruct((B, S, D), jnp.float32),
        scratch_types=(
            [pltpu.VMEM((_CRW, D), jnp.float32)] * 10
            + [pltpu.SemaphoreType.DMA] * 18
        ),
    )(_sc_body)
    return k(X, pos_table[:S])


kernel = _kernel_sc


# speedup vs baseline: 1.0125x; 1.0125x over previous
import functools

import jax
import jax.numpy as jnp
from jax import lax
from jax.experimental import pallas as pl
from jax.experimental.pallas import tpu as pltpu
from jax.experimental.pallas import tpu_sc as plsc


def _add_block(x_ref, pos_ref, o_ref):
    o_ref[...] = x_ref[...] + pos_ref[...]


def _kernel_tc(X, pos_table):
    B, S, D = X.shape
    bs = 2048
    grid = (S // bs, B)
    out = pl.pallas_call(
        _add_block,
        grid=grid,
        in_specs=[
            pl.BlockSpec((1, bs, D), lambda s, b: (b, s, 0)),
            pl.BlockSpec((bs, D), lambda s, b: (s, 0)),
        ],
        out_specs=pl.BlockSpec((1, bs, D), lambda s, b: (b, s, 0)),
        out_shape=jax.ShapeDtypeStruct((B, S, D), X.dtype),
    )(X, pos_table[:S])
    return out


_NC, _NS, _NL = 2, 16, 16
_NW = _NC * _NS
_CRW = 8
_NCH = 2


def _sc_body(x_hbm, pos_hbm, o_hbm, *scratch):
    nx = 4 * _NCH
    xbs = scratch[0:nx]
    pbs = scratch[nx:nx + _NCH]
    sxl = scratch[nx + _NCH:2 * nx + _NCH]
    sxs = scratch[2 * nx + _NCH:3 * nx + _NCH]
    spl = scratch[3 * nx + _NCH:3 * nx + 2 * _NCH]

    w = lax.axis_index("c") * _NS + lax.axis_index("s")
    B, S, D = x_hbm.shape
    rows = S // _NW
    r0 = w * rows
    nsc = rows // _CRW

    def xload(sc, b, xb, sem):
        r = r0 + sc * _CRW
        pltpu.make_async_copy(x_hbm.at[b, pl.ds(r, _CRW), :], xb, sem).start()

    def xload_wait(xb, sem):
        pltpu.make_async_copy(x_hbm.at[0, pl.ds(r0, _CRW), :], xb, sem).wait()

    def pload(sc, pb, sem):
        r = r0 + sc * _CRW
        pltpu.make_async_copy(pos_hbm.at[pl.ds(r, _CRW), :], pb, sem).start()

    def pload_wait(pb, sem):
        pltpu.make_async_copy(pos_hbm.at[pl.ds(r0, _CRW), :], pb, sem).wait()

    def xstore(sc, b, xb, sem):
        r = r0 + sc * _CRW
        pltpu.make_async_copy(xb, o_hbm.at[b, pl.ds(r, _CRW), :], sem).start()

    def xstore_wait(xb, sem):
        pltpu.make_async_copy(xb, o_hbm.at[0, pl.ds(r0, _CRW), :], sem).wait()

    def compute(xb, pb):
        @plsc.parallel_loop(0, _CRW * D, step=_NL, unroll=8)
        def add(v):
            r = v // D
            sl = pl.ds(v % D, _NL)
            xb[r, sl] = xb[r, sl] + pb[r, sl]

    def half_pass(sc, pre, base, pb, psem):
        pload_wait(pb, psem)
        for b in range(4):
            xload_wait(xbs[base + b], sxl[base + b])
            compute(xbs[base + b], pb)
            xstore(sc, b, xbs[base + b], sxs[base + b])
        pload(pre, pb, psem)
        for b in range(4):
            xstore_wait(xbs[base + b], sxs[base + b])
            xload(pre, b, xbs[base + b], sxl[base + b])

    for j in range(_NCH):
        pload(j, pbs[j], spl[j])
        for b in range(4):
            xload(j, b, xbs[4 * j + b], sxl[4 * j + b])

    def body(i, carry):
        sc0 = i * _NCH
        for j in range(_NCH):
            pre = lax.min(sc0 + _NCH + j, nsc - _NCH + j)
            half_pass(sc0 + j, pre, 4 * j, pbs[j], spl[j])
        return carry

    lax.fori_loop(0, nsc // _NCH, body, 0)

    for j in range(nx):
        xload_wait(xbs[j], sxl[j])
    for j in range(_NCH):
        pload_wait(pbs[j], spl[j])


def _kernel_sc(X, pos_table):
    B, S, D = X.shape
    mesh = plsc.VectorSubcoreMesh(core_axis_name="c", subcore_axis_name="s")
    k = functools.partial(
        pl.kernel,
        mesh=mesh,
        out_type=jax.ShapeDtypeStruct((B, S, D), jnp.float32),
        scratch_types=(
            [pltpu.VMEM((_CRW, D), jnp.float32)] * (5 * _NCH)
            + [pltpu.SemaphoreType.DMA] * (9 * _NCH)
        ),
    )(_sc_body)
    return k(X, pos_table[:S])


kernel = _kernel_sc
